# Initial kernel scaffold; baseline (speedup 1.0000x reference)
import jax
import jax.numpy as jnp
from jax.experimental import pallas as pl


def kernel(x, edge_index, batch, W1, a_src1, a_dst1, b1, W2, a_src2, a_dst2, b2, lin1_W, lin1_b, lin2_W, lin2_b):
    raise NotImplementedError("write your pallas kernel here")


# speedup vs baseline: 15.5149x; 15.5149x over previous
import functools

import jax
import jax.numpy as jnp
from jax import lax
from jax.experimental import pallas as pl
from jax.experimental.pallas import tpu as pltpu
from jax.experimental.pallas import tpu_sc as plsc

N = 100000
NP = 100352
D = 32
HID = 128

NT = 16
HALF = NP // 2
SEG = HALF + 8
B = 128
ROWS_PER_TILE = HALF // NT

E_TOTAL = 1700000
EB = NT * B
EP = ((E_TOTAL + EB - 1) // EB) * EB
STEPS = EP // (NT * B)
CHUNK = EP // NT

R_TC = 512
POOL_R = 800


def _dense1_body(x_ref, w_ref, a2t_ref, h_ref, alar_ref):
    h = jnp.dot(x_ref[...], w_ref[...], preferred_element_type=jnp.float32)
    h_ref[...] = h
    alar_ref[...] = lax.dot_general(
        a2t_ref[...], h, (((1,), (1,)), ((), ())),
        preferred_element_type=jnp.float32)


def _dense23_body(p_ref, b_ref, w_ref, a2t_ref, h_ref, alar_ref):
    x = jnp.maximum(p_ref[...] + b_ref[...], 0.0)
    h = jnp.dot(x, w_ref[...], preferred_element_type=jnp.float32)
    h_ref[...] = h
    alar_ref[...] = lax.dot_general(
        a2t_ref[...], h, (((1,), (1,)), ((), ())),
        preferred_element_type=jnp.float32)


def _dense1(xp, w, a2t):
    d_in = xp.shape[1]
    return pl.pallas_call(
        _dense1_body,
        grid=(NP // R_TC,),
        in_specs=[
            pl.BlockSpec((R_TC, d_in), lambda i: (i, 0)),
            pl.BlockSpec((d_in, D), lambda i: (0, 0)),
            pl.BlockSpec((2, D), lambda i: (0, 0)),
        ],
        out_specs=[
            pl.BlockSpec((R_TC, D), lambda i: (i, 0)),
            pl.BlockSpec((2, R_TC), lambda i: (0, i)),
        ],
        out_shape=[
            jax.ShapeDtypeStruct((NP, D), jnp.float32),
            jax.ShapeDtypeStruct((2, NP), jnp.float32),
        ],
    )(xp, w, a2t)


def _dense23(prev, brow, w, a2t):
    return pl.pallas_call(
        _dense23_body,
        grid=(NP // R_TC,),
        in_specs=[
            pl.BlockSpec((R_TC, D), lambda i: (i, 0)),
            pl.BlockSpec((1, D), lambda i: (0, 0)),
            pl.BlockSpec((D, D), lambda i: (0, 0)),
            pl.BlockSpec((2, D), lambda i: (0, 0)),
        ],
        out_specs=[
            pl.BlockSpec((R_TC, D), lambda i: (i, 0)),
            pl.BlockSpec((2, R_TC), lambda i: (0, i)),
        ],
        out_shape=[
            jax.ShapeDtypeStruct((NP, D), jnp.float32),
            jax.ShapeDtypeStruct((2, NP), jnp.float32),
        ],
    )(prev, brow, w, a2t)


def _pool_body(p_ref, b_ref, w1_ref, b1_ref, w2t_ref, o_ref, acc_ref):
    i = pl.program_id(0)
    x = jnp.maximum(p_ref[...] + b_ref[...], 0.0)
    part = jnp.max(x, axis=0, keepdims=True)

    @pl.when(i == 0)
    def _():
        acc_ref[...] = part

    @pl.when(i > 0)
    def _():
        acc_ref[...] = jnp.maximum(acc_ref[...], part)

    @pl.when(i == pl.num_programs(0) - 1)
    def _():
        m = jnp.broadcast_to(acc_ref[...], (8, D))
        z = jnp.maximum(
            jnp.dot(m, w1_ref[...], preferred_element_type=jnp.float32)
            + b1_ref[...], 0.0)
        o_ref[...] = jnp.sum(z[0:1, :] * w2t_ref[...], keepdims=True)


def _pool_mlp(out3, brow, w1, b1row, w2t, lin2_b):
    val = pl.pallas_call(
        _pool_body,
        grid=(N // POOL_R,),
        in_specs=[
            pl.BlockSpec((POOL_R, D), lambda i: (i, 0)),
            pl.BlockSpec((1, D), lambda i: (0, 0)),
            pl.BlockSpec((D, HID), lambda i: (0, 0)),
            pl.BlockSpec((1, HID), lambda i: (0, 0)),
            pl.BlockSpec((1, HID), lambda i: (0, 0)),
        ],
        out_specs=pl.BlockSpec((1, 1), lambda i: (0, 0)),
        out_shape=jax.ShapeDtypeStruct((1, 1), jnp.float32),
        scratch_shapes=[pltpu.VMEM((1, D), jnp.float32)],
    )(out3, brow, w1, b1row, w2t)
    return val + lin2_b[0]


_MESH = plsc.VectorSubcoreMesh(core_axis_name="c", subcore_axis_name="s")


@functools.partial(
    pl.kernel,
    mesh=_MESH,
    compiler_params=pltpu.CompilerParams(use_tc_tiling_on_sc=False),
    out_type=jax.ShapeDtypeStruct((NP, D), jnp.float32),
    scratch_types=[
        pltpu.VMEM_SHARED((NP,), jnp.float32),
        pltpu.VMEM_SHARED((NP,), jnp.float32),
        pltpu.VMEM_SHARED((SEG,), jnp.float32),
        pltpu.VMEM_SHARED((SEG, D), jnp.float32),
        pltpu.VMEM((B,), jnp.int32),
        pltpu.VMEM((B,), jnp.int32),
        pltpu.VMEM((B,), jnp.int32),
        pltpu.VMEM((B,), jnp.float32),
        pltpu.VMEM((B,), jnp.float32),
        pltpu.VMEM((B,), jnp.float32),
        pltpu.VMEM((B,), jnp.float32),
        pltpu.VMEM((B, D), jnp.float32),
        pltpu.VMEM((NP // NT,), jnp.float32),
    ],
)
def _edge_kernel(src_hbm, dst_hbm, alar_hbm, h_hbm, out_hbm,
                 al_sh, ar_sh, s_sh, acc_sh,
                 srcv, dstv, dlv, alsv, arsv, sv, exv, rows, stage):
    c = lax.axis_index("c")
    t = lax.axis_index("s")
    base = c * HALF
    cbase = t * CHUNK

    w16 = NP // NT
    off = t * w16
    pltpu.sync_copy(alar_hbm.at[0, pl.ds(off, w16)], stage)
    pltpu.sync_copy(stage, al_sh.at[pl.ds(off, w16)])
    pltpu.sync_copy(alar_hbm.at[1, pl.ds(off, w16)], stage)
    pltpu.sync_copy(stage, ar_sh.at[pl.ds(off, w16)])

    @pl.loop(0, B)
    def _(i):
        rows[i, pl.ds(0, 16)] = jnp.zeros((16,), jnp.float32)
        rows[i, pl.ds(16, 16)] = jnp.zeros((16,), jnp.float32)

    @pl.loop(0, B, step=16)
    def _(i):
        exv[pl.ds(i, 16)] = jnp.zeros((16,), jnp.float32)

    zoff = t * ROWS_PER_TILE
    for k in range(ROWS_PER_TILE // B):
        pltpu.sync_copy(rows, acc_sh.at[pl.ds(zoff + k * B, B)])
        pltpu.sync_copy(exv, s_sh.at[pl.ds(zoff + k * B, B)])
    rem = ROWS_PER_TILE % B
    if rem:
        pltpu.sync_copy(rows.at[pl.ds(0, rem)],
                        acc_sh.at[pl.ds(zoff + ROWS_PER_TILE - rem, rem)])
        pltpu.sync_copy(exv.at[pl.ds(0, rem)],
                        s_sh.at[pl.ds(zoff + ROWS_PER_TILE - rem, rem)])

    @pl.when(t == 0)
    def _():
        pltpu.sync_copy(rows.at[pl.ds(0, 8)], acc_sh.at[pl.ds(HALF, 8)])
        pltpu.sync_copy(exv.at[pl.ds(0, 8)], s_sh.at[pl.ds(HALF, 8)])

    plsc.subcore_barrier()

    @pl.loop(0, STEPS)
    def _(step):
        eoff = cbase + step * B
        pltpu.sync_copy(src_hbm.at[pl.ds(eoff, B)], srcv)
        pltpu.sync_copy(dst_hbm.at[pl.ds(eoff, B)], dstv)
        pltpu.sync_copy(al_sh.at[srcv], alsv)
        pltpu.sync_copy(ar_sh.at[dstv], arsv)

        @pl.loop(0, B, step=16)
        def _(i):
            sl = pl.ds(i, 16)
            e = alsv[sl] + arsv[sl]
            e = jnp.maximum(e, e * 0.2)
            exv[sl] = jnp.exp(e)
            dl = dstv[sl] - base
            ok = (dl >= 0) & (dl < HALF)
            dlv[sl] = jnp.where(ok, dl, HALF)

        pltpu.sync_copy(exv, s_sh.at[dlv], add=True)

    plsc.subcore_barrier()

    @pl.loop(0, STEPS)
    def _(step):
        eoff = cbase + step * B
        pltpu.sync_copy(src_hbm.at[pl.ds(eoff, B)], srcv)
        pltpu.sync_copy(dst_hbm.at[pl.ds(eoff, B)], dstv)
        pltpu.sync_copy(al_sh.at[srcv], alsv)
        pltpu.sync_copy(ar_sh.at[dstv], arsv)

        @pl.loop(0, B, step=16)
        def _(i):
            sl = pl.ds(i, 16)
            dl = dstv[sl] - base
            ok = (dl >= 0) & (dl < HALF)
            dlv[sl] = jnp.where(ok, dl, HALF)

        pltpu.sync_copy(s_sh.at[dlv], sv)
        pltpu.sync_copy(h_hbm.at[srcv], rows)

        @pl.loop(0, B, step=16)
        def _(i):
            sl = pl.ds(i, 16)
            e = alsv[sl] + arsv[sl]
            e = jnp.maximum(e, e * 0.2)
            exv[sl] = jnp.exp(e) / (sv[sl] + 1e-16)

        @pl.loop(0, B, step=16)
        def _(g):
            av = exv[pl.ds(g, 16)]
            for j in range(16):
                a = av[j]
                rows[g + j, pl.ds(0, 16)] = rows[g + j, pl.ds(0, 16)] * a
                rows[g + j, pl.ds(16, 16)] = rows[g + j, pl.ds(16, 16)] * a

        pltpu.sync_copy(rows, acc_sh.at[dlv], add=True)

    plsc.subcore_barrier()

    woff = t * ROWS_PER_TILE
    for k in range(ROWS_PER_TILE // B):
        pltpu.sync_copy(acc_sh.at[pl.ds(woff + k * B, B)], rows)
        pltpu.sync_copy(rows, out_hbm.at[pl.ds(base + woff + k * B, B)])
    if rem:
        pltpu.sync_copy(acc_sh.at[pl.ds(woff + ROWS_PER_TILE - rem, rem)],
                        rows.at[pl.ds(0, rem)])
        pltpu.sync_copy(rows.at[pl.ds(0, rem)],
                        out_hbm.at[pl.ds(base + woff + ROWS_PER_TILE - rem,
                                         rem)])


def kernel(x, edge_index, batch, W1, a_src1, a_dst1, b1, W2, a_src2, a_dst2,
           b2, lin1_W, lin1_b, lin2_W, lin2_b):
    n = x.shape[0]
    loops = jnp.arange(n, dtype=edge_index.dtype)
    src = jnp.concatenate([edge_index[0], loops])
    dst = jnp.concatenate([edge_index[1], loops])
    pad = EP - src.shape[0]
    sent = jnp.full((pad,), n, jnp.int32)
    src = jnp.concatenate([src, sent])
    dst = jnp.concatenate([dst, sent])

    xp = jnp.zeros((NP, x.shape[1]), jnp.float32).at[:n].set(x)
    a2t_1 = jnp.stack([a_src1, a_dst1])
    a2t_2 = jnp.stack([a_src2, a_dst2])
    b1row = b1.reshape(1, D)
    b2row = b2.reshape(1, D)

    h, alar = _dense1(xp, W1, a2t_1)
    out1 = _edge_kernel(src, dst, alar, h)
    h, alar = _dense23(out1, b1row, W2, a2t_2)
    out2 = _edge_kernel(src, dst, alar, h)
    h, alar = _dense23(out2, b2row, W2, a2t_2)
    out3 = _edge_kernel(src, dst, alar, h)

    return _pool_mlp(out3, b2row, lin1_W, lin1_b.reshape(1, HID),
                     lin2_W.reshape(1, HID), lin2_b)


# speedup vs baseline: 22.2686x; 1.4353x over previous
import functools

import jax
import jax.numpy as jnp
from jax import lax
from jax.experimental import pallas as pl
from jax.experimental.pallas import tpu as pltpu
from jax.experimental.pallas import tpu_sc as plsc

N = 100000
NP = 100352
D = 32
HID = 128

NT = 16
HALF = NP // 2
SEG = HALF + 8
B = 256
ROWS_PER_TILE = HALF // NT

E_TOTAL = 1700000
EB = NT * B
EP = ((E_TOTAL + EB - 1) // EB) * EB
STEPS = EP // (NT * B)
CHUNK = EP // NT

R_TC = 512
POOL_R = 800


def _dense1_body(x_ref, w_ref, a2t_ref, h_ref, alar_ref):
    h = jnp.dot(x_ref[...], w_ref[...], preferred_element_type=jnp.float32)
    h_ref[...] = h
    alar_ref[...] = lax.dot_general(
        a2t_ref[...], h, (((1,), (1,)), ((), ())),
        preferred_element_type=jnp.float32)


def _dense23_body(p_ref, b_ref, w_ref, a2t_ref, h_ref, alar_ref):
    x = jnp.maximum(p_ref[...] + b_ref[...], 0.0)
    h = jnp.dot(x, w_ref[...], preferred_element_type=jnp.float32)
    h_ref[...] = h
    alar_ref[...] = lax.dot_general(
        a2t_ref[...], h, (((1,), (1,)), ((), ())),
        preferred_element_type=jnp.float32)


def _dense1(xp, w, a2t):
    d_in = xp.shape[1]
    return pl.pallas_call(
        _dense1_body,
        grid=(NP // R_TC,),
        in_specs=[
            pl.BlockSpec((R_TC, d_in), lambda i: (i, 0)),
            pl.BlockSpec((d_in, D), lambda i: (0, 0)),
            pl.BlockSpec((2, D), lambda i: (0, 0)),
        ],
        out_specs=[
            pl.BlockSpec((R_TC, D), lambda i: (i, 0)),
            pl.BlockSpec((2, R_TC), lambda i: (0, i)),
        ],
        out_shape=[
            jax.ShapeDtypeStruct((NP, D), jnp.float32),
            jax.ShapeDtypeStruct((2, NP), jnp.float32),
        ],
    )(xp, w, a2t)


def _dense23(prev, brow, w, a2t):
    return pl.pallas_call(
        _dense23_body,
        grid=(NP // R_TC,),
        in_specs=[
            pl.BlockSpec((R_TC, D), lambda i: (i, 0)),
            pl.BlockSpec((1, D), lambda i: (0, 0)),
            pl.BlockSpec((D, D), lambda i: (0, 0)),
            pl.BlockSpec((2, D), lambda i: (0, 0)),
        ],
        out_specs=[
            pl.BlockSpec((R_TC, D), lambda i: (i, 0)),
            pl.BlockSpec((2, R_TC), lambda i: (0, i)),
        ],
        out_shape=[
            jax.ShapeDtypeStruct((NP, D), jnp.float32),
            jax.ShapeDtypeStruct((2, NP), jnp.float32),
        ],
    )(prev, brow, w, a2t)


def _pool_body(p_ref, b_ref, w1_ref, b1_ref, w2t_ref, o_ref, acc_ref):
    i = pl.program_id(0)
    x = jnp.maximum(p_ref[...] + b_ref[...], 0.0)
    part = jnp.max(x, axis=0, keepdims=True)

    @pl.when(i == 0)
    def _():
        acc_ref[...] = part

    @pl.when(i > 0)
    def _():
        acc_ref[...] = jnp.maximum(acc_ref[...], part)

    @pl.when(i == pl.num_programs(0) - 1)
    def _():
        m = jnp.broadcast_to(acc_ref[...], (8, D))
        z = jnp.maximum(
            jnp.dot(m, w1_ref[...], preferred_element_type=jnp.float32)
            + b1_ref[...], 0.0)
        o_ref[...] = jnp.sum(z[0:1, :] * w2t_ref[...], keepdims=True)


def _pool_mlp(out3, brow, w1, b1row, w2t, lin2_b):
    val = pl.pallas_call(
        _pool_body,
        grid=(N // POOL_R,),
        in_specs=[
            pl.BlockSpec((POOL_R, D), lambda i: (i, 0)),
            pl.BlockSpec((1, D), lambda i: (0, 0)),
            pl.BlockSpec((D, HID), lambda i: (0, 0)),
            pl.BlockSpec((1, HID), lambda i: (0, 0)),
            pl.BlockSpec((1, HID), lambda i: (0, 0)),
        ],
        out_specs=pl.BlockSpec((1, 1), lambda i: (0, 0)),
        out_shape=jax.ShapeDtypeStruct((1, 1), jnp.float32),
        scratch_shapes=[pltpu.VMEM((1, D), jnp.float32)],
    )(out3, brow, w1, b1row, w2t)
    return val + lin2_b[0]


_MESH = plsc.VectorSubcoreMesh(core_axis_name="c", subcore_axis_name="s")


@functools.partial(
    pl.kernel,
    mesh=_MESH,
    compiler_params=pltpu.CompilerParams(use_tc_tiling_on_sc=False),
    out_type=jax.ShapeDtypeStruct((NP, D), jnp.float32),
    scratch_types=[
        pltpu.VMEM_SHARED((NP,), jnp.float32),
        pltpu.VMEM_SHARED((NP,), jnp.float32),
        pltpu.VMEM_SHARED((SEG,), jnp.float32),
        pltpu.VMEM_SHARED((SEG, D), jnp.float32),
        pltpu.VMEM((B,), jnp.int32),
        pltpu.VMEM((B,), jnp.int32),
        pltpu.VMEM((B,), jnp.int32),
        pltpu.VMEM((B,), jnp.float32),
        pltpu.VMEM((B,), jnp.float32),
        pltpu.VMEM((B,), jnp.float32),
        pltpu.VMEM((B, D), jnp.float32),
        pltpu.SemaphoreType.DMA,
        pltpu.SemaphoreType.DMA,
        pltpu.SemaphoreType.DMA,
    ],
)
def _edge_kernel(src_hbm, dst_hbm, alar_hbm, h_hbm, out_hbm,
                 al_sh, ar_sh, s_sh, acc_sh,
                 srcv, dstv, dlv, alsv, arsv, sv, rows, sem0, sem1, sem2):
    c = lax.axis_index("c")
    t = lax.axis_index("s")
    base = c * HALF
    cbase = t * CHUNK

    w16 = NP // NT
    off = t * w16
    srem = w16 % B

    @pl.loop(0, w16 // B)
    def _(k):
        o2 = off + k * B
        ha = pltpu.async_copy(alar_hbm.at[0, pl.ds(o2, B)], alsv, sem0)
        hb = pltpu.async_copy(alar_hbm.at[1, pl.ds(o2, B)], arsv, sem1)
        ha.wait()
        hb.wait()
        ha = pltpu.async_copy(alsv, al_sh.at[pl.ds(o2, B)], sem0)
        hb = pltpu.async_copy(arsv, ar_sh.at[pl.ds(o2, B)], sem1)
        ha.wait()
        hb.wait()

    if srem:
        o2 = off + (w16 // B) * B
        pltpu.sync_copy(alar_hbm.at[0, pl.ds(o2, srem)],
                        alsv.at[pl.ds(0, srem)])
        pltpu.sync_copy(alsv.at[pl.ds(0, srem)], al_sh.at[pl.ds(o2, srem)])
        pltpu.sync_copy(alar_hbm.at[1, pl.ds(o2, srem)],
                        arsv.at[pl.ds(0, srem)])
        pltpu.sync_copy(arsv.at[pl.ds(0, srem)], ar_sh.at[pl.ds(o2, srem)])

    @pl.loop(0, B)
    def _(i):
        rows[i, pl.ds(0, 16)] = jnp.zeros((16,), jnp.float32)
        rows[i, pl.ds(16, 16)] = jnp.zeros((16,), jnp.float32)

    @pl.loop(0, B, step=16)
    def _(i):
        sv[pl.ds(i, 16)] = jnp.zeros((16,), jnp.float32)

    zoff = t * ROWS_PER_TILE
    for k in range(ROWS_PER_TILE // B):
        pltpu.sync_copy(rows, acc_sh.at[pl.ds(zoff + k * B, B)])
        pltpu.sync_copy(sv, s_sh.at[pl.ds(zoff + k * B, B)])
    rem = ROWS_PER_TILE % B
    if rem:
        pltpu.sync_copy(rows.at[pl.ds(0, rem)],
                        acc_sh.at[pl.ds(zoff + ROWS_PER_TILE - rem, rem)])
        pltpu.sync_copy(sv.at[pl.ds(0, rem)],
                        s_sh.at[pl.ds(zoff + ROWS_PER_TILE - rem, rem)])

    @pl.when(t == 0)
    def _():
        pltpu.sync_copy(rows.at[pl.ds(0, 8)], acc_sh.at[pl.ds(HALF, 8)])
        pltpu.sync_copy(sv.at[pl.ds(0, 8)], s_sh.at[pl.ds(HALF, 8)])

    plsc.subcore_barrier()

    @pl.loop(0, STEPS)
    def _(step):
        eoff = cbase + step * B
        h0 = pltpu.async_copy(src_hbm.at[pl.ds(eoff, B)], srcv, sem0)
        h1 = pltpu.async_copy(dst_hbm.at[pl.ds(eoff, B)], dstv, sem1)
        h0.wait()
        h1.wait()
        g0 = pltpu.async_copy(al_sh.at[srcv], alsv, sem0)
        g1 = pltpu.async_copy(ar_sh.at[dstv], arsv, sem1)
        g0.wait()
        g1.wait()

        @pl.loop(0, B, step=16)
        def _(i):
            sl = pl.ds(i, 16)
            e = alsv[sl] + arsv[sl]
            e = jnp.maximum(e, e * 0.2)
            alsv[sl] = jnp.exp(e)
            dl = dstv[sl] - base
            ok = (dl >= 0) & (dl < HALF)
            dlv[sl] = jnp.where(ok, dl, HALF)

        pltpu.sync_copy(alsv, s_sh.at[dlv], add=True)

    plsc.subcore_barrier()

    @pl.loop(0, STEPS)
    def _(step):
        eoff = cbase + step * B
        h0 = pltpu.async_copy(src_hbm.at[pl.ds(eoff, B)], srcv, sem0)
        h1 = pltpu.async_copy(dst_hbm.at[pl.ds(eoff, B)], dstv, sem1)
        h0.wait()
        h1.wait()
        g0 = pltpu.async_copy(al_sh.at[srcv], alsv, sem0)
        g1 = pltpu.async_copy(ar_sh.at[dstv], arsv, sem1)
        g2 = pltpu.async_copy(h_hbm.at[srcv], rows, sem2)

        @pl.loop(0, B, step=16)
        def _(i):
            sl = pl.ds(i, 16)
            dl = dstv[sl] - base
            ok = (dl >= 0) & (dl < HALF)
            dlv[sl] = jnp.where(ok, dl, HALF)

        g0.wait()
        g1.wait()
        gs = pltpu.async_copy(s_sh.at[dlv], sv, sem0)
        gs.wait()

        @pl.loop(0, B, step=16)
        def _(i):
            sl = pl.ds(i, 16)
            e = alsv[sl] + arsv[sl]
            e = jnp.maximum(e, e * 0.2)
            sv[sl] = jnp.exp(e) / (sv[sl] + 1e-16)

        g2.wait()

        @pl.loop(0, B, step=16)
        def _(g):
            av = sv[pl.ds(g, 16)]
            for j in range(16):
                a = av[j]
                rows[g + j, pl.ds(0, 16)] = rows[g + j, pl.ds(0, 16)] * a
                rows[g + j, pl.ds(16, 16)] = rows[g + j, pl.ds(16, 16)] * a

        pltpu.sync_copy(rows, acc_sh.at[dlv], add=True)

    plsc.subcore_barrier()

    woff = t * ROWS_PER_TILE
    for k in range(ROWS_PER_TILE // B):
        pltpu.sync_copy(acc_sh.at[pl.ds(woff + k * B, B)], rows)
        pltpu.sync_copy(rows, out_hbm.at[pl.ds(base + woff + k * B, B)])
    if rem:
        pltpu.sync_copy(acc_sh.at[pl.ds(woff + ROWS_PER_TILE - rem, rem)],
                        rows.at[pl.ds(0, rem)])
        pltpu.sync_copy(rows.at[pl.ds(0, rem)],
                        out_hbm.at[pl.ds(base + woff + ROWS_PER_TILE - rem,
                                         rem)])


def kernel(x, edge_index, batch, W1, a_src1, a_dst1, b1, W2, a_src2, a_dst2,
           b2, lin1_W, lin1_b, lin2_W, lin2_b):
    n = x.shape[0]
    loops = jnp.arange(n, dtype=edge_index.dtype)
    src = jnp.concatenate([edge_index[0], loops])
    dst = jnp.concatenate([edge_index[1], loops])
    pad = EP - src.shape[0]
    sent = jnp.full((pad,), n, jnp.int32)
    src = jnp.concatenate([src, sent])
    dst = jnp.concatenate([dst, sent])

    xp = jnp.zeros((NP, x.shape[1]), jnp.float32).at[:n].set(x)
    a2t_1 = jnp.stack([a_src1, a_dst1])
    a2t_2 = jnp.stack([a_src2, a_dst2])
    b1row = b1.reshape(1, D)
    b2row = b2.reshape(1, D)

    h, alar = _dense1(xp, W1, a2t_1)
    out1 = _edge_kernel(src, dst, alar, h)
    h, alar = _dense23(out1, b1row, W2, a2t_2)
    out2 = _edge_kernel(src, dst, alar, h)
    h, alar = _dense23(out2, b2row, W2, a2t_2)
    out3 = _edge_kernel(src, dst, alar, h)

    return _pool_mlp(out3, b2row, lin1_W, lin1_b.reshape(1, HID),
                     lin2_W.reshape(1, HID), lin2_b)


# speedup vs baseline: 30.2757x; 1.3596x over previous
import functools

import jax
import jax.numpy as jnp
from jax import lax
from jax.experimental import pallas as pl
from jax.experimental.pallas import tpu as pltpu
from jax.experimental.pallas import tpu_sc as plsc

N = 100000
NP = 100352
D = 32
HID = 128

NT = 16
HALF = NP // 2
SEG = HALF + 8
B = 256
ROWS_PER_TILE = HALF // NT

E_TOTAL = 1700000
EB = NT * B
EP = ((E_TOTAL + EB - 1) // EB) * EB
STEPS = EP // (NT * B)
CHUNK = EP // NT

R_TC = 512
POOL_R = 800


def _dense1_body(x_ref, w_ref, a2t_ref, h_ref, alar_ref):
    h = jnp.dot(x_ref[...], w_ref[...], preferred_element_type=jnp.float32)
    h_ref[...] = h
    alar_ref[...] = lax.dot_general(
        a2t_ref[...], h, (((1,), (1,)), ((), ())),
        preferred_element_type=jnp.float32)


def _dense23_body(p_ref, b_ref, w_ref, a2t_ref, h_ref, alar_ref):
    x = jnp.maximum(p_ref[...] + b_ref[...], 0.0)
    h = jnp.dot(x, w_ref[...], preferred_element_type=jnp.float32)
    h_ref[...] = h
    alar_ref[...] = lax.dot_general(
        a2t_ref[...], h, (((1,), (1,)), ((), ())),
        preferred_element_type=jnp.float32)


def _dense1(xp, w, a2t):
    d_in = xp.shape[1]
    return pl.pallas_call(
        _dense1_body,
        grid=(NP // R_TC,),
        in_specs=[
            pl.BlockSpec((R_TC, d_in), lambda i: (i, 0)),
            pl.BlockSpec((d_in, D), lambda i: (0, 0)),
            pl.BlockSpec((2, D), lambda i: (0, 0)),
        ],
        out_specs=[
            pl.BlockSpec((R_TC, D), lambda i: (i, 0)),
            pl.BlockSpec((2, R_TC), lambda i: (0, i)),
        ],
        out_shape=[
            jax.ShapeDtypeStruct((NP, D), jnp.float32),
            jax.ShapeDtypeStruct((2, NP), jnp.float32),
        ],
    )(xp, w, a2t)


def _dense23(prev, brow, w, a2t):
    return pl.pallas_call(
        _dense23_body,
        grid=(NP // R_TC,),
        in_specs=[
            pl.BlockSpec((R_TC, D), lambda i: (i, 0)),
            pl.BlockSpec((1, D), lambda i: (0, 0)),
            pl.BlockSpec((D, D), lambda i: (0, 0)),
            pl.BlockSpec((2, D), lambda i: (0, 0)),
        ],
        out_specs=[
            pl.BlockSpec((R_TC, D), lambda i: (i, 0)),
            pl.BlockSpec((2, R_TC), lambda i: (0, i)),
        ],
        out_shape=[
            jax.ShapeDtypeStruct((NP, D), jnp.float32),
            jax.ShapeDtypeStruct((2, NP), jnp.float32),
        ],
    )(prev, brow, w, a2t)


def _pool_body(p_ref, b_ref, w1_ref, b1_ref, w2t_ref, o_ref, acc_ref):
    i = pl.program_id(0)
    x = jnp.maximum(p_ref[...] + b_ref[...], 0.0)
    part = jnp.max(x, axis=0, keepdims=True)

    @pl.when(i == 0)
    def _():
        acc_ref[...] = part

    @pl.when(i > 0)
    def _():
        acc_ref[...] = jnp.maximum(acc_ref[...], part)

    @pl.when(i == pl.num_programs(0) - 1)
    def _():
        m = jnp.broadcast_to(acc_ref[...], (8, D))
        z = jnp.maximum(
            jnp.dot(m, w1_ref[...], preferred_element_type=jnp.float32)
            + b1_ref[...], 0.0)
        o_ref[...] = jnp.sum(z[0:1, :] * w2t_ref[...], keepdims=True)


def _pool_mlp(out3, brow, w1, b1row, w2t, lin2_b):
    val = pl.pallas_call(
        _pool_body,
        grid=(N // POOL_R,),
        in_specs=[
            pl.BlockSpec((POOL_R, D), lambda i: (i, 0)),
            pl.BlockSpec((1, D), lambda i: (0, 0)),
            pl.BlockSpec((D, HID), lambda i: (0, 0)),
            pl.BlockSpec((1, HID), lambda i: (0, 0)),
            pl.BlockSpec((1, HID), lambda i: (0, 0)),
        ],
        out_specs=pl.BlockSpec((1, 1), lambda i: (0, 0)),
        out_shape=jax.ShapeDtypeStruct((1, 1), jnp.float32),
        scratch_shapes=[pltpu.VMEM((1, D), jnp.float32)],
    )(out3, brow, w1, b1row, w2t)
    return val + lin2_b[0]


_MESH = plsc.VectorSubcoreMesh(core_axis_name="c", subcore_axis_name="s")


@functools.partial(
    pl.kernel,
    mesh=_MESH,
    compiler_params=pltpu.CompilerParams(use_tc_tiling_on_sc=False),
    out_type=jax.ShapeDtypeStruct((NP, D), jnp.float32),
    scratch_types=[
        pltpu.VMEM_SHARED((NP,), jnp.float32),
        pltpu.VMEM_SHARED((NP,), jnp.float32),
        pltpu.VMEM_SHARED((SEG,), jnp.float32),
        pltpu.VMEM_SHARED((SEG, D), jnp.float32),
        pltpu.VMEM((B,), jnp.int32),
        pltpu.VMEM((B,), jnp.int32),
        pltpu.VMEM((B,), jnp.int32),
        pltpu.VMEM((B,), jnp.int32),
        pltpu.VMEM((B,), jnp.int32),
        pltpu.VMEM((B,), jnp.float32),
        pltpu.VMEM((B,), jnp.float32),
        pltpu.VMEM((B,), jnp.float32),
        pltpu.VMEM((B, D), jnp.float32),
        pltpu.SemaphoreType.DMA,
        pltpu.SemaphoreType.DMA,
        pltpu.SemaphoreType.DMA,
    ],
)
def _edge_kernel(src_hbm, dst_hbm, alar_hbm, h_hbm, out_hbm,
                 al_sh, ar_sh, s_sh, acc_sh,
                 srcv0, srcv1, dstv0, dstv1, dlv, alsv, arsv, sv, rows,
                 sem0, sem1, sem2):
    c = lax.axis_index("c")
    t = lax.axis_index("s")
    base = c * HALF
    cbase = t * CHUNK

    w16 = NP // NT
    off = t * w16
    srem = w16 % B

    @pl.loop(0, w16 // B)
    def _(k):
        o2 = off + k * B
        ha = pltpu.async_copy(alar_hbm.at[0, pl.ds(o2, B)], alsv, sem0)
        hb = pltpu.async_copy(alar_hbm.at[1, pl.ds(o2, B)], arsv, sem1)
        ha.wait()
        hb.wait()
        ha = pltpu.async_copy(alsv, al_sh.at[pl.ds(o2, B)], sem0)
        hb = pltpu.async_copy(arsv, ar_sh.at[pl.ds(o2, B)], sem1)
        ha.wait()
        hb.wait()

    if srem:
        o2 = off + (w16 // B) * B
        pltpu.sync_copy(alar_hbm.at[0, pl.ds(o2, srem)],
                        alsv.at[pl.ds(0, srem)])
        pltpu.sync_copy(alsv.at[pl.ds(0, srem)], al_sh.at[pl.ds(o2, srem)])
        pltpu.sync_copy(alar_hbm.at[1, pl.ds(o2, srem)],
                        arsv.at[pl.ds(0, srem)])
        pltpu.sync_copy(arsv.at[pl.ds(0, srem)], ar_sh.at[pl.ds(o2, srem)])

    @pl.loop(0, B)
    def _(i):
        rows[i, pl.ds(0, 16)] = jnp.zeros((16,), jnp.float32)
        rows[i, pl.ds(16, 16)] = jnp.zeros((16,), jnp.float32)

    @pl.loop(0, B, step=16)
    def _(i):
        sv[pl.ds(i, 16)] = jnp.zeros((16,), jnp.float32)

    zoff = t * ROWS_PER_TILE
    for k in range(ROWS_PER_TILE // B):
        pltpu.sync_copy(rows, acc_sh.at[pl.ds(zoff + k * B, B)])
        pltpu.sync_copy(sv, s_sh.at[pl.ds(zoff + k * B, B)])
    rem = ROWS_PER_TILE % B
    if rem:
        pltpu.sync_copy(rows.at[pl.ds(0, rem)],
                        acc_sh.at[pl.ds(zoff + ROWS_PER_TILE - rem, rem)])
        pltpu.sync_copy(sv.at[pl.ds(0, rem)],
                        s_sh.at[pl.ds(zoff + ROWS_PER_TILE - rem, rem)])

    @pl.when(t == 0)
    def _():
        pltpu.sync_copy(rows.at[pl.ds(0, 8)], acc_sh.at[pl.ds(HALF, 8)])
        pltpu.sync_copy(sv.at[pl.ds(0, 8)], s_sh.at[pl.ds(HALF, 8)])

    plsc.subcore_barrier()

    h0 = pltpu.async_copy(src_hbm.at[pl.ds(cbase, B)], srcv0, sem0)
    h1 = pltpu.async_copy(dst_hbm.at[pl.ds(cbase, B)], dstv0, sem1)

    @pl.loop(0, STEPS // 2)
    def _(u):
        for b, (sb, db, so, do) in enumerate(
                ((srcv0, dstv0, srcv1, dstv1),
                 (srcv1, dstv1, srcv0, dstv0))):
            step = u * 2 + b
            pltpu.make_async_copy(src_hbm.at[pl.ds(cbase, B)], sb,
                                  sem0).wait()
            pltpu.make_async_copy(dst_hbm.at[pl.ds(cbase, B)], db,
                                  sem1).wait()
            g2 = pltpu.async_copy(h_hbm.at[sb], rows, sem2)

            @pl.when(step + 1 < STEPS)
            def _():
                eoff2 = cbase + (step + 1) * B
                pltpu.async_copy(src_hbm.at[pl.ds(eoff2, B)], so, sem0)
                pltpu.async_copy(dst_hbm.at[pl.ds(eoff2, B)], do, sem1)

            pltpu.sync_copy(al_sh.at[sb], alsv)
            pltpu.sync_copy(ar_sh.at[db], arsv)

            @pl.loop(0, B, step=16)
            def _(i):
                sl = pl.ds(i, 16)
                e = alsv[sl] + arsv[sl]
                e = jnp.maximum(e, e * 0.2)
                alsv[sl] = jnp.exp(e)
                dl = db[sl] - base
                ok = (dl >= 0) & (dl < HALF)
                dlv[sl] = jnp.where(ok, dl, HALF)

            pltpu.sync_copy(alsv, s_sh.at[dlv], add=True)
            g2.wait()

            @pl.loop(0, B, step=16)
            def _(g):
                av = alsv[pl.ds(g, 16)]
                for j in range(16):
                    a = av[j]
                    rows[g + j, pl.ds(0, 16)] = rows[g + j, pl.ds(0, 16)] * a
                    rows[g + j, pl.ds(16, 16)] = (
                        rows[g + j, pl.ds(16, 16)] * a)

            pltpu.sync_copy(rows, acc_sh.at[dlv], add=True)

    plsc.subcore_barrier()

    woff = t * ROWS_PER_TILE
    nchunks = ROWS_PER_TILE // B + (1 if rem else 0)
    for k in range(nchunks):
        cw = B if k < ROWS_PER_TILE // B else rem
        co = woff + k * B
        pltpu.sync_copy(acc_sh.at[pl.ds(co, cw)], rows.at[pl.ds(0, cw)])
        pltpu.sync_copy(s_sh.at[pl.ds(co, cw)], sv.at[pl.ds(0, cw)])

        @pl.loop(0, cw, step=16)
        def _(i):
            sl = pl.ds(i, 16)
            sv[sl] = 1.0 / (sv[sl] + 1e-16)

        @pl.loop(0, cw, step=16)
        def _(g):
            av = sv[pl.ds(g, 16)]
            for j in range(16):
                a = av[j]
                rows[g + j, pl.ds(0, 16)] = rows[g + j, pl.ds(0, 16)] * a
                rows[g + j, pl.ds(16, 16)] = rows[g + j, pl.ds(16, 16)] * a

        pltpu.sync_copy(rows.at[pl.ds(0, cw)],
                        out_hbm.at[pl.ds(base + co, cw)])


def kernel(x, edge_index, batch, W1, a_src1, a_dst1, b1, W2, a_src2, a_dst2,
           b2, lin1_W, lin1_b, lin2_W, lin2_b):
    n = x.shape[0]
    loops = jnp.arange(n, dtype=edge_index.dtype)
    src = jnp.concatenate([edge_index[0], loops])
    dst = jnp.concatenate([edge_index[1], loops])
    pad = EP - src.shape[0]
    sent = jnp.full((pad,), n, jnp.int32)
    src = jnp.concatenate([src, sent])
    dst = jnp.concatenate([dst, sent])

    xp = jnp.zeros((NP, x.shape[1]), jnp.float32).at[:n].set(x)
    a2t_1 = jnp.stack([a_src1, a_dst1])
    a2t_2 = jnp.stack([a_src2, a_dst2])
    b1row = b1.reshape(1, D)
    b2row = b2.reshape(1, D)

    h, alar = _dense1(xp, W1, a2t_1)
    out1 = _edge_kernel(src, dst, alar, h)
    h, alar = _dense23(out1, b1row, W2, a2t_2)
    out2 = _edge_kernel(src, dst, alar, h)
    h, alar = _dense23(out2, b2row, W2, a2t_2)
    out3 = _edge_kernel(src, dst, alar, h)

    return _pool_mlp(out3, b2row, lin1_W, lin1_b.reshape(1, HID),
                     lin2_W.reshape(1, HID), lin2_b)
